# baseline (speedup 1.0000x reference)
import jax
import jax.numpy as jnp
from jax import lax
from jax.experimental import pallas as pl
from jax.experimental.pallas import tpu as pltpu
from jax.experimental.pallas import tpu_sc as plsc

N = 10000
E = 320000
D = 128

NC = 2
NS = 16
NW = NC * NS

B = 128
EPT = 10240
NCHUNK = EPT // B
NP = 10240
RPT = NP // NS


def _agg_body(h_hbm, src_hbm, dst_hbm, zeros_hbm, out_hbm,
              src_v, dst_v, rows_v, agg_sh, gsem0):
    c = lax.axis_index("c")
    s = lax.axis_index("s")
    w = c * NS + s

    pltpu.sync_copy(zeros_hbm.at[pl.ds(s * RPT, RPT)],
                    agg_sh.at[pl.ds(s * RPT, RPT)])
    pltpu.sync_copy(src_hbm.at[w], src_v)
    pltpu.sync_copy(dst_hbm.at[w], dst_v)
    plsc.subcore_barrier()

    @pl.loop(0, NCHUNK)
    def _(j):
        pltpu.async_copy(h_hbm.at[src_v.at[j]], rows_v, gsem0).wait()
        pltpu.sync_copy(rows_v, agg_sh.at[dst_v.at[j]], add=True)

    plsc.subcore_barrier()
    pltpu.sync_copy(agg_sh.at[pl.ds(s * RPT, RPT)],
                    out_hbm.at[c, pl.ds(s * RPT, RPT)])


_agg_call = pl.kernel(
    _agg_body,
    out_type=jax.ShapeDtypeStruct((NC, NP, D), jnp.float32),
    mesh=plsc.VectorSubcoreMesh(core_axis_name="c", subcore_axis_name="s",
                                num_cores=NC, num_subcores=NS),
    scratch_types=[
        pltpu.VMEM((NCHUNK, B), jnp.int32),
        pltpu.VMEM((NCHUNK, B), jnp.int32),
        pltpu.VMEM((B, D), jnp.float32),
        pltpu.VMEM_SHARED((NP, D), jnp.float32),
        pltpu.SemaphoreType.DMA,
    ],
)


ROW_BLK = 400


def _mm_body(h_ref, r0_ref, r1_ref, w_ref, b_ref, o_ref):
    rst = h_ref[...] + r0_ref[...] + r1_ref[...]
    o_ref[...] = (
        jnp.dot(rst, w_ref[...], preferred_element_type=jnp.float32)
        + b_ref[...]
    )


def _mm_call(parts, h, w, b):
    return pl.pallas_call(
        _mm_body,
        grid=(N // ROW_BLK,),
        in_specs=[
            pl.BlockSpec((ROW_BLK, D), lambda i: (i, 0)),
            pl.BlockSpec((ROW_BLK, D), lambda i: (i, 0)),
            pl.BlockSpec((ROW_BLK, D), lambda i: (i, 0)),
            pl.BlockSpec((D, D), lambda i: (0, 0)),
            pl.BlockSpec((1, D), lambda i: (0, 0)),
        ],
        out_specs=pl.BlockSpec((ROW_BLK, D), lambda i: (i, 0)),
        out_shape=jax.ShapeDtypeStruct((N, D), jnp.float32),
    )(h, parts[0, :N], parts[1, :N], w, b.reshape(1, D))


def kernel(h, edge_index, W0, W1, W2, W3, W4, b0, b1, b2, b3, b4):
    Ws = [W0, W1, W2, W3, W4]
    bs = [b0, b1, b2, b3, b4]
    src2 = edge_index[0].reshape(NW, E // NW)
    dst2 = edge_index[1].reshape(NW, E // NW)
    pad = EPT - E // NW
    src = jnp.pad(src2, ((0, 0), (0, pad))).reshape(NW, NCHUNK, B)
    dummy = N + (jnp.arange(pad, dtype=jnp.int32)[None, :]
                 + 37 * jnp.arange(NW, dtype=jnp.int32)[:, None]) % (NP - N)
    dst = jnp.concatenate([dst2, dummy], axis=1).reshape(NW, NCHUNK, B)
    zeros = jnp.zeros((NP, D), dtype=jnp.float32)
    for i in range(5):
        parts = _agg_call(h, src, dst, zeros)
        h = _mm_call(parts, h, Ws[i], bs[i])
    return h


# speedup vs baseline: 2.1991x; 2.1991x over previous
import jax
import jax.numpy as jnp
from jax import lax
from jax.experimental import pallas as pl
from jax.experimental.pallas import tpu as pltpu
from jax.experimental.pallas import tpu_sc as plsc

N = 10000
E = 320000
D = 128

NC = 2
NS = 16
NW = NC * NS

B = 100
EPT = 10000
NCHUNK = EPT // B
NP = 10240
RPT = NP // NS


def _agg_body(h_hbm, src_hbm, dst_hbm, zeros_hbm, out_hbm,
              src_v, dst_v, rows_v, agg_sh, gsem0):
    c = lax.axis_index("c")
    s = lax.axis_index("s")
    w = c * NS + s

    pltpu.sync_copy(zeros_hbm.at[pl.ds(s * RPT, RPT)],
                    agg_sh.at[pl.ds(s * RPT, RPT)])
    pltpu.sync_copy(src_hbm.at[w], src_v)
    pltpu.sync_copy(dst_hbm.at[w], dst_v)
    plsc.subcore_barrier()

    @pl.loop(0, NCHUNK)
    def _(j):
        pltpu.async_copy(h_hbm.at[src_v.at[j]], rows_v, gsem0).wait()
        pltpu.sync_copy(rows_v, agg_sh.at[dst_v.at[j]], add=True)

    plsc.subcore_barrier()
    pltpu.sync_copy(agg_sh.at[pl.ds(s * RPT, RPT)],
                    out_hbm.at[c, pl.ds(s * RPT, RPT)])


_agg_call = pl.kernel(
    _agg_body,
    out_type=jax.ShapeDtypeStruct((NC, NP, D), jnp.float32),
    mesh=plsc.VectorSubcoreMesh(core_axis_name="c", subcore_axis_name="s",
                                num_cores=NC, num_subcores=NS),
    scratch_types=[
        pltpu.VMEM((NCHUNK, B), jnp.int32),
        pltpu.VMEM((NCHUNK, B), jnp.int32),
        pltpu.VMEM((B, D), jnp.float32),
        pltpu.VMEM_SHARED((NP, D), jnp.float32),
        pltpu.SemaphoreType.DMA,
    ],
)


ROW_BLK = 400


def _mm_body(h_ref, r0_ref, r1_ref, w_ref, b_ref, o_ref):
    rst = h_ref[...] + r0_ref[...] + r1_ref[...]
    o_ref[...] = (
        jnp.dot(rst, w_ref[...], preferred_element_type=jnp.float32)
        + b_ref[...]
    )


def _mm_call(parts, h, w, b):
    return pl.pallas_call(
        _mm_body,
        grid=(N // ROW_BLK,),
        in_specs=[
            pl.BlockSpec((ROW_BLK, D), lambda i: (i, 0)),
            pl.BlockSpec((ROW_BLK, D), lambda i: (i, 0)),
            pl.BlockSpec((ROW_BLK, D), lambda i: (i, 0)),
            pl.BlockSpec((D, D), lambda i: (0, 0)),
            pl.BlockSpec((1, D), lambda i: (0, 0)),
        ],
        out_specs=pl.BlockSpec((ROW_BLK, D), lambda i: (i, 0)),
        out_shape=jax.ShapeDtypeStruct((N, D), jnp.float32),
    )(h, parts[0, :N], parts[1, :N], w, b.reshape(1, D))


def kernel(h, edge_index, W0, W1, W2, W3, W4, b0, b1, b2, b3, b4):
    Ws = [W0, W1, W2, W3, W4]
    bs = [b0, b1, b2, b3, b4]
    src = edge_index[0].reshape(NW, NCHUNK, B)
    dst = edge_index[1].reshape(NW, NCHUNK, B)
    zeros = jnp.zeros((NP, D), dtype=jnp.float32)
    for i in range(5):
        parts = _agg_call(h, src, dst, zeros)
        h = _mm_call(parts, h, Ws[i], bs[i])
    return h


# speedup vs baseline: 2.6004x; 1.1825x over previous
import jax
import jax.numpy as jnp
from jax import lax
from jax.experimental import pallas as pl
from jax.experimental.pallas import tpu as pltpu
from jax.experimental.pallas import tpu_sc as plsc

N = 10000
E = 320000
D = 128

NC = 2
NS = 16
NW = NC * NS

B = 100
EPT = 10000
NCHUNK = EPT // B
NPASS = 2
PCH = NCHUNK // NPASS
NBUF = 2
NP = 10240
RPT = NP // NS


def _agg_body(h_hbm, src_hbm, dst_hbm, zeros_hbm, out_hbm,
              src_v, dst_v, rows_v, agg_sh, gsem0, gsem1):
    gsems = [gsem0, gsem1]
    c = lax.axis_index("c")
    s = lax.axis_index("s")
    w = c * NS + s

    pltpu.sync_copy(zeros_hbm.at[pl.ds(s * RPT, RPT)],
                    agg_sh.at[pl.ds(s * RPT, RPT)])
    plsc.subcore_barrier()

    for p in range(NPASS):
        pltpu.sync_copy(src_hbm.at[w * NPASS + p], src_v)
        pltpu.sync_copy(dst_hbm.at[w * NPASS + p], dst_v)

        @pl.loop(0, PCH // NBUF)
        def _(b):
            descs = [
                pltpu.async_copy(h_hbm.at[src_v.at[b * NBUF + k]],
                                 rows_v.at[k], gsems[k])
                for k in range(NBUF)
            ]
            for k in range(NBUF):
                descs[k].wait()
                pltpu.sync_copy(rows_v.at[k],
                                agg_sh.at[dst_v.at[b * NBUF + k]], add=True)

    plsc.subcore_barrier()
    pltpu.sync_copy(agg_sh.at[pl.ds(s * RPT, RPT)],
                    out_hbm.at[c, pl.ds(s * RPT, RPT)])


_agg_call = pl.kernel(
    _agg_body,
    out_type=jax.ShapeDtypeStruct((NC, NP, D), jnp.float32),
    mesh=plsc.VectorSubcoreMesh(core_axis_name="c", subcore_axis_name="s",
                                num_cores=NC, num_subcores=NS),
    scratch_types=[
        pltpu.VMEM((PCH, B), jnp.int32),
        pltpu.VMEM((PCH, B), jnp.int32),
        pltpu.VMEM((NBUF, B, D), jnp.float32),
        pltpu.VMEM_SHARED((NP, D), jnp.float32),
        pltpu.SemaphoreType.DMA,
        pltpu.SemaphoreType.DMA,
    ],
)


ROW_BLK = 400


def _mm_body(h_ref, r0_ref, r1_ref, w_ref, b_ref, o_ref):
    rst = h_ref[...] + r0_ref[...] + r1_ref[...]
    o_ref[...] = (
        jnp.dot(rst, w_ref[...], preferred_element_type=jnp.float32)
        + b_ref[...]
    )


def _mm_call(parts, h, w, b):
    return pl.pallas_call(
        _mm_body,
        grid=(N // ROW_BLK,),
        in_specs=[
            pl.BlockSpec((ROW_BLK, D), lambda i: (i, 0)),
            pl.BlockSpec((ROW_BLK, D), lambda i: (i, 0)),
            pl.BlockSpec((ROW_BLK, D), lambda i: (i, 0)),
            pl.BlockSpec((D, D), lambda i: (0, 0)),
            pl.BlockSpec((1, D), lambda i: (0, 0)),
        ],
        out_specs=pl.BlockSpec((ROW_BLK, D), lambda i: (i, 0)),
        out_shape=jax.ShapeDtypeStruct((N, D), jnp.float32),
    )(h, parts[0, :N], parts[1, :N], w, b.reshape(1, D))


def kernel(h, edge_index, W0, W1, W2, W3, W4, b0, b1, b2, b3, b4):
    Ws = [W0, W1, W2, W3, W4]
    bs = [b0, b1, b2, b3, b4]
    src = edge_index[0].reshape(NW * NPASS, PCH, B)
    dst = edge_index[1].reshape(NW * NPASS, PCH, B)
    zeros = jnp.zeros((NP, D), dtype=jnp.float32)
    for i in range(5):
        parts = _agg_call(h, src, dst, zeros)
        h = _mm_call(parts, h, Ws[i], bs[i])
    return h


# speedup vs baseline: 2.6658x; 1.0251x over previous
import jax
import jax.numpy as jnp
from jax import lax
from jax.experimental import pallas as pl
from jax.experimental.pallas import tpu as pltpu
from jax.experimental.pallas import tpu_sc as plsc

N = 10000
E = 320000
D = 128

NC = 2
NS = 16
NW = NC * NS

B = 125
EPT = 10000
NCHUNK = EPT // B
NPASS = 2
PCH = NCHUNK // NPASS
NBUF = 2
NP = 10240
RPT = NP // NS


def _agg_body(h_hbm, src_hbm, dst_hbm, zeros_hbm, out_hbm,
              src_v, dst_v, rows_v, agg_sh, gsem0, gsem1):
    gsems = [gsem0, gsem1]
    c = lax.axis_index("c")
    s = lax.axis_index("s")
    w = c * NS + s

    pltpu.sync_copy(zeros_hbm.at[pl.ds(s * RPT, RPT)],
                    agg_sh.at[pl.ds(s * RPT, RPT)])
    plsc.subcore_barrier()

    for p in range(NPASS):
        pltpu.sync_copy(src_hbm.at[w * NPASS + p], src_v)
        pltpu.sync_copy(dst_hbm.at[w * NPASS + p], dst_v)

        @pl.loop(0, PCH // NBUF)
        def _(b):
            descs = [
                pltpu.async_copy(h_hbm.at[src_v.at[b * NBUF + k]],
                                 rows_v.at[k], gsems[k])
                for k in range(NBUF)
            ]
            for k in range(NBUF):
                descs[k].wait()
                pltpu.sync_copy(rows_v.at[k],
                                agg_sh.at[dst_v.at[b * NBUF + k]], add=True)

    plsc.subcore_barrier()
    pltpu.sync_copy(agg_sh.at[pl.ds(s * RPT, RPT)],
                    out_hbm.at[c, pl.ds(s * RPT, RPT)])


_agg_call = pl.kernel(
    _agg_body,
    out_type=jax.ShapeDtypeStruct((NC, NP, D), jnp.float32),
    mesh=plsc.VectorSubcoreMesh(core_axis_name="c", subcore_axis_name="s",
                                num_cores=NC, num_subcores=NS),
    scratch_types=[
        pltpu.VMEM((PCH, B), jnp.int32),
        pltpu.VMEM((PCH, B), jnp.int32),
        pltpu.VMEM((NBUF, B, D), jnp.float32),
        pltpu.VMEM_SHARED((NP, D), jnp.float32),
        pltpu.SemaphoreType.DMA,
        pltpu.SemaphoreType.DMA,
    ],
)


ROW_BLK = 400


def _mm_body(h_ref, r0_ref, r1_ref, w_ref, b_ref, o_ref):
    rst = h_ref[...] + r0_ref[...] + r1_ref[...]
    o_ref[...] = (
        jnp.dot(rst, w_ref[...], preferred_element_type=jnp.float32)
        + b_ref[...]
    )


def _mm_call(parts, h, w, b):
    return pl.pallas_call(
        _mm_body,
        grid=(N // ROW_BLK,),
        in_specs=[
            pl.BlockSpec((ROW_BLK, D), lambda i: (i, 0)),
            pl.BlockSpec((ROW_BLK, D), lambda i: (i, 0)),
            pl.BlockSpec((ROW_BLK, D), lambda i: (i, 0)),
            pl.BlockSpec((D, D), lambda i: (0, 0)),
            pl.BlockSpec((1, D), lambda i: (0, 0)),
        ],
        out_specs=pl.BlockSpec((ROW_BLK, D), lambda i: (i, 0)),
        out_shape=jax.ShapeDtypeStruct((N, D), jnp.float32),
    )(h, parts[0, :N], parts[1, :N], w, b.reshape(1, D))


def kernel(h, edge_index, W0, W1, W2, W3, W4, b0, b1, b2, b3, b4):
    Ws = [W0, W1, W2, W3, W4]
    bs = [b0, b1, b2, b3, b4]
    src = edge_index[0].reshape(NW * NPASS, PCH, B)
    dst = edge_index[1].reshape(NW * NPASS, PCH, B)
    zeros = jnp.zeros((NP, D), dtype=jnp.float32)
    for i in range(5):
        parts = _agg_call(h, src, dst, zeros)
        h = _mm_call(parts, h, Ws[i], bs[i])
    return h


# speedup vs baseline: 2.6957x; 1.0112x over previous
import jax
import jax.numpy as jnp
from jax import lax
from jax.experimental import pallas as pl
from jax.experimental.pallas import tpu as pltpu
from jax.experimental.pallas import tpu_sc as plsc

N = 10000
E = 320000
D = 128

NC = 2
NS = 16
NW = NC * NS

B = 125
EPT = 10000
NCHUNK = EPT // B
NPASS = 2
PCH = NCHUNK // NPASS
NBUF = 2
NP = 10240
RPT = NP // NS


def _agg_body(h_hbm, src_hbm, dst_hbm, zeros_hbm, out_hbm,
              src_v, dst_v, rows_v, agg_sh, gsem0, gsem1, ssem0, ssem1):
    gsems = [gsem0, gsem1]
    ssems = [ssem0, ssem1]
    c = lax.axis_index("c")
    s = lax.axis_index("s")
    w = c * NS + s

    pltpu.sync_copy(zeros_hbm.at[pl.ds(s * RPT, RPT)],
                    agg_sh.at[pl.ds(s * RPT, RPT)])
    plsc.subcore_barrier()

    for p in range(NPASS):
        pltpu.sync_copy(src_hbm.at[w * NPASS + p], src_v)
        pltpu.sync_copy(dst_hbm.at[w * NPASS + p], dst_v)

        @pl.loop(0, PCH // NBUF)
        def _(b):
            descs = [
                pltpu.async_copy(h_hbm.at[src_v.at[b * NBUF + k]],
                                 rows_v.at[k], gsems[k])
                for k in range(NBUF)
            ]
            sdescs = []
            for k in range(NBUF):
                descs[k].wait()
                sdescs.append(
                    pltpu.async_copy(rows_v.at[k],
                                     agg_sh.at[dst_v.at[b * NBUF + k]],
                                     ssems[k], add=True))
            for d in sdescs:
                d.wait()

    plsc.subcore_barrier()
    pltpu.sync_copy(agg_sh.at[pl.ds(s * RPT, RPT)],
                    out_hbm.at[c, pl.ds(s * RPT, RPT)])


_agg_call = pl.kernel(
    _agg_body,
    out_type=jax.ShapeDtypeStruct((NC, NP, D), jnp.float32),
    mesh=plsc.VectorSubcoreMesh(core_axis_name="c", subcore_axis_name="s",
                                num_cores=NC, num_subcores=NS),
    scratch_types=[
        pltpu.VMEM((PCH, B), jnp.int32),
        pltpu.VMEM((PCH, B), jnp.int32),
        pltpu.VMEM((NBUF, B, D), jnp.float32),
        pltpu.VMEM_SHARED((NP, D), jnp.float32),
        pltpu.SemaphoreType.DMA,
        pltpu.SemaphoreType.DMA,
        pltpu.SemaphoreType.DMA,
        pltpu.SemaphoreType.DMA,
    ],
)


ROW_BLK = 400


def _mm_body(h_ref, r0_ref, r1_ref, w_ref, b_ref, o_ref):
    rst = h_ref[...] + r0_ref[...] + r1_ref[...]
    o_ref[...] = (
        jnp.dot(rst, w_ref[...], preferred_element_type=jnp.float32)
        + b_ref[...]
    )


def _mm_call(parts, h, w, b):
    return pl.pallas_call(
        _mm_body,
        grid=(N // ROW_BLK,),
        in_specs=[
            pl.BlockSpec((ROW_BLK, D), lambda i: (i, 0)),
            pl.BlockSpec((ROW_BLK, D), lambda i: (i, 0)),
            pl.BlockSpec((ROW_BLK, D), lambda i: (i, 0)),
            pl.BlockSpec((D, D), lambda i: (0, 0)),
            pl.BlockSpec((1, D), lambda i: (0, 0)),
        ],
        out_specs=pl.BlockSpec((ROW_BLK, D), lambda i: (i, 0)),
        out_shape=jax.ShapeDtypeStruct((N, D), jnp.float32),
    )(h, parts[0, :N], parts[1, :N], w, b.reshape(1, D))


def kernel(h, edge_index, W0, W1, W2, W3, W4, b0, b1, b2, b3, b4):
    Ws = [W0, W1, W2, W3, W4]
    bs = [b0, b1, b2, b3, b4]
    src = edge_index[0].reshape(NW * NPASS, PCH, B)
    dst = edge_index[1].reshape(NW * NPASS, PCH, B)
    zeros = jnp.zeros((NP, D), dtype=jnp.float32)
    for i in range(5):
        parts = _agg_call(h, src, dst, zeros)
        h = _mm_call(parts, h, Ws[i], bs[i])
    return h


# speedup vs baseline: 2.8047x; 1.0404x over previous
import jax
import jax.numpy as jnp
from jax import lax
from jax.experimental import pallas as pl
from jax.experimental.pallas import tpu as pltpu
from jax.experimental.pallas import tpu_sc as plsc

N = 10000
E = 320000
D = 128

NC = 2
NS = 16
NW = NC * NS

B = 125
EPT = 10000
NCHUNK = EPT // B
NPASS = 2
PCH = NCHUNK // NPASS
NBUF = 2
NP = 10240
RPT = NP // NS


def _agg_body(h_hbm, src_hbm, dst_hbm, zeros_hbm, out_hbm,
              src_v, dst_v, rows_v, agg_sh, gsem0, gsem1, ssem0, ssem1):
    gsems = [gsem0, gsem1]
    ssems = [ssem0, ssem1]
    c = lax.axis_index("c")
    s = lax.axis_index("s")
    w = c * NS + s

    pltpu.sync_copy(zeros_hbm.at[pl.ds(s * RPT, RPT)],
                    agg_sh.at[pl.ds(s * RPT, RPT)])
    plsc.subcore_barrier()

    for p in range(NPASS):
        pltpu.sync_copy(src_hbm.at[w * NPASS + p], src_v)
        pltpu.sync_copy(dst_hbm.at[w * NPASS + p], dst_v)

        @pl.loop(0, PCH // NBUF)
        def _(b):
            descs = [
                pltpu.async_copy(h_hbm.at[src_v.at[b * NBUF + k]],
                                 rows_v.at[k], gsems[k])
                for k in range(NBUF)
            ]
            sdescs = []
            for k in range(NBUF):
                descs[k].wait()
                sdescs.append(
                    pltpu.async_copy(rows_v.at[k],
                                     agg_sh.at[dst_v.at[b * NBUF + k]],
                                     ssems[k], add=True))
            for d in sdescs:
                d.wait()

    plsc.subcore_barrier()
    pltpu.sync_copy(agg_sh.at[pl.ds(s * RPT, RPT)],
                    out_hbm.at[c, pl.ds(s * RPT, RPT)])


_agg_call = pl.kernel(
    _agg_body,
    out_type=jax.ShapeDtypeStruct((NC, NP, D), jnp.float32),
    mesh=plsc.VectorSubcoreMesh(core_axis_name="c", subcore_axis_name="s",
                                num_cores=NC, num_subcores=NS),
    scratch_types=[
        pltpu.VMEM((PCH, B), jnp.int32),
        pltpu.VMEM((PCH, B), jnp.int32),
        pltpu.VMEM((NBUF, B, D), jnp.float32),
        pltpu.VMEM_SHARED((NP, D), jnp.float32),
        pltpu.SemaphoreType.DMA,
        pltpu.SemaphoreType.DMA,
        pltpu.SemaphoreType.DMA,
        pltpu.SemaphoreType.DMA,
    ],
)


ROW_BLK = 400


def _mm_body(h_ref, r0_ref, r1_ref, w_ref, b_ref, o_ref):
    rst = h_ref[...] + r0_ref[0] + r1_ref[0]
    o_ref[...] = (
        jnp.dot(rst, w_ref[...], preferred_element_type=jnp.float32)
        + b_ref[...]
    )


def _mm_call(parts, h, w, b):
    return pl.pallas_call(
        _mm_body,
        grid=(N // ROW_BLK,),
        in_specs=[
            pl.BlockSpec((ROW_BLK, D), lambda i: (i, 0)),
            pl.BlockSpec((1, ROW_BLK, D), lambda i: (0, i, 0)),
            pl.BlockSpec((1, ROW_BLK, D), lambda i: (1, i, 0)),
            pl.BlockSpec((D, D), lambda i: (0, 0)),
            pl.BlockSpec((1, D), lambda i: (0, 0)),
        ],
        out_specs=pl.BlockSpec((ROW_BLK, D), lambda i: (i, 0)),
        out_shape=jax.ShapeDtypeStruct((N, D), jnp.float32),
    )(h, parts, parts, w, b.reshape(1, D))


def kernel(h, edge_index, W0, W1, W2, W3, W4, b0, b1, b2, b3, b4):
    Ws = [W0, W1, W2, W3, W4]
    bs = [b0, b1, b2, b3, b4]
    src = edge_index[0].reshape(NW * NPASS, PCH, B)
    dst = edge_index[1].reshape(NW * NPASS, PCH, B)
    zeros = jnp.zeros((NP, D), dtype=jnp.float32)
    for i in range(5):
        parts = _agg_call(h, src, dst, zeros)
        h = _mm_call(parts, h, Ws[i], bs[i])
    return h
